# baseline (speedup 1.0000x reference)
import functools

import jax
import jax.numpy as jnp
from jax import lax
from jax.experimental import pallas as pl
from jax.experimental.pallas import tpu as pltpu
from jax.experimental.pallas import tpu_sc as plsc

VOCAB = 1000000
ACT_NUM = 1000
EMB = 64
B = 16384
W = 2 * EMB

NC = 2
NS = 16
L = 16
NW = NC * NS
BPW = B // NW
NQ = EMB // L
CH = 256
NCH = BPW // CH
GPC = CH // L

_mesh = plsc.VectorSubcoreMesh(core_axis_name="c", subcore_axis_name="s")


@functools.partial(
    pl.kernel,
    out_type=jax.ShapeDtypeStruct((B,), jnp.float32),
    mesh=_mesh,
    scratch_types=[
        pltpu.VMEM((BPW,), jnp.int32),
        pltpu.VMEM((BPW,), jnp.int32),
        pltpu.VMEM((BPW,), jnp.int32),
        [pltpu.VMEM((CH,), jnp.int32)] * NCH,
        [pltpu.VMEM((CH,), jnp.int32)] * NCH,
        [pltpu.VMEM((CH,), jnp.int32)] * NCH,
        pltpu.VMEM((CH, W), jnp.float32),
        pltpu.VMEM((CH, W), jnp.float32),
        pltpu.VMEM((CH, W), jnp.float32),
        pltpu.VMEM((EMB,), jnp.float32),
        pltpu.VMEM((BPW,), jnp.float32),
        pltpu.VMEM((L, L), jnp.float32),
        pltpu.SemaphoreType.DMA,
    ],
    compiler_params=pltpu.CompilerParams(needs_layout_passes=False),
)
def _transe_sc(head_hbm, tail_hbm, act_hbm, ct_hbm, at_hbm, c_hbm, out_hbm,
               hidx_v, tidx_v, aidx_v, hmaj_v, tmaj_v, amaj_v,
               h_v, t_v, a_v, c_v, out_v, pbuf_v, sem):
    wid = lax.axis_index("s") * NC + lax.axis_index("c")
    base = pl.multiple_of(wid * BPW, BPW)

    pltpu.sync_copy(head_hbm.at[pl.ds(base, BPW)], hidx_v)
    pltpu.sync_copy(tail_hbm.at[pl.ds(base, BPW)], tidx_v)
    pltpu.sync_copy(act_hbm.at[pl.ds(base, BPW)], aidx_v)
    pltpu.sync_copy(c_hbm, c_v)

    for k in range(BPW // L):
        sl = pl.ds(k * L, L)
        ch, off = k // GPC, (k % GPC) * L
        osl = pl.ds(off, L)
        hmaj_v[ch][osl] = lax.shift_right_logical(hidx_v[sl], 1)
        tmaj_v[ch][osl] = lax.shift_right_logical(tidx_v[sl], 1)
        amaj_v[ch][osl] = lax.shift_right_logical(aidx_v[sl], 1)

    cs = [c_v[pl.ds(q * L, L)] for q in range(NQ)]
    lane = jnp.arange(L, dtype=jnp.int32)
    inv = jnp.float32(1.0 / EMB)
    one = jnp.int32(1)

    for ch in range(NCH):
        cp_h = pltpu.async_copy(ct_hbm.at[hmaj_v[ch]], h_v, sem)
        cp_t = pltpu.async_copy(ct_hbm.at[tmaj_v[ch]], t_v, sem)
        cp_a = pltpu.async_copy(at_hbm.at[amaj_v[ch]], a_v, sem)
        cp_h.wait()
        cp_t.wait()
        cp_a.wait()

        def grp(g, carry):
            row0 = pl.multiple_of(g * L, L)
            rsl = pl.ds(pl.multiple_of(ch * CH + row0, L), L)
            phv = lax.shift_left(lax.bitwise_and(hidx_v[rsl], one), 6)
            ptv = lax.shift_left(lax.bitwise_and(tidx_v[rsl], one), 6)
            pav = lax.shift_left(lax.bitwise_and(aidx_v[rsl], one), 6)
            for i in range(L):
                j = row0 + i
                ph = phv[i]
                pt = ptv[i]
                pa = pav[i]
                d = None
                for q in range(NQ):
                    o = q * L
                    hq = h_v[j, pl.ds(pl.multiple_of(ph + o, L), L)]
                    tq = t_v[j, pl.ds(pl.multiple_of(pt + o, L), L)]
                    aq = a_v[j, pl.ds(pl.multiple_of(pa + o, L), L)]
                    dq = jnp.abs(hq + aq - tq + cs[q])
                    d = dq if d is None else d + dq
                plsc.store_scatter(
                    pbuf_v, [lane, jnp.full((L,), i, jnp.int32)], d)
            acc = pbuf_v[0, :]
            for rr in range(1, L):
                acc = acc + pbuf_v[rr, :]
            out_v[pl.ds(pl.multiple_of(ch * CH + row0, L), L)] = acc * inv
            return carry

        lax.fori_loop(0, GPC, grp, 0)

    pltpu.sync_copy(out_v, out_hbm.at[pl.ds(base, BPW)])


def kernel(data, concept_table, act_table, begin, end):
    head = data[:, 0].astype(jnp.int32)
    act = data[:, 1].astype(jnp.int32)
    tail = data[:, 2].astype(jnp.int32)
    cvec = (begin - end).reshape(EMB).astype(jnp.float32)
    ct2 = concept_table.reshape(VOCAB // 2, W)
    at2 = act_table.reshape(ACT_NUM // 2, W)
    return _transe_sc(head, tail, act, ct2, at2, cvec)


# speedup vs baseline: 1.4450x; 1.4450x over previous
import functools

import jax
import jax.numpy as jnp
from jax import lax
from jax.experimental import pallas as pl
from jax.experimental.pallas import tpu as pltpu
from jax.experimental.pallas import tpu_sc as plsc

VOCAB = 1000000
ACT_NUM = 1000
EMB = 64
B = 16384
W = 2 * EMB
SL = 8

NC = 2
NS = 16
L = 16
NW = NC * NS
BPW = B // NW
NQ = EMB // L
G = BPW // L

_mesh = plsc.VectorSubcoreMesh(core_axis_name="c", subcore_axis_name="s")


@functools.partial(
    pl.kernel,
    out_type=jax.ShapeDtypeStruct((B,), jnp.float32),
    mesh=_mesh,
    scratch_types=[
        pltpu.VMEM((BPW,), jnp.int32),
        pltpu.VMEM((BPW,), jnp.int32),
        pltpu.VMEM((BPW,), jnp.int32),
        [pltpu.VMEM((L, SL, EMB), jnp.float32)] * 2,
        [pltpu.VMEM((L, SL, EMB), jnp.float32)] * 2,
        [pltpu.VMEM((L, SL, EMB), jnp.float32)] * 2,
        pltpu.VMEM((EMB,), jnp.float32),
        pltpu.VMEM((BPW,), jnp.float32),
        pltpu.VMEM((L, L), jnp.float32),
        [pltpu.SemaphoreType.DMA] * 2,
    ],
    compiler_params=pltpu.CompilerParams(needs_layout_passes=False),
)
def _transe_sc(head_hbm, tail_hbm, act_hbm, ct_hbm, at_hbm, c_hbm, out_hbm,
               hidx_v, tidx_v, aidx_v, h_v, t_v, a_v, c_v, out_v,
               pbuf_v, sems):
    wid = lax.axis_index("s") * NC + lax.axis_index("c")
    base = pl.multiple_of(wid * BPW, BPW)

    pltpu.sync_copy(head_hbm.at[pl.ds(base, BPW)], hidx_v)
    pltpu.sync_copy(tail_hbm.at[pl.ds(base, BPW)], tidx_v)
    pltpu.sync_copy(act_hbm.at[pl.ds(base, BPW)], aidx_v)
    pltpu.sync_copy(c_hbm, c_v)

    cs = [c_v[pl.ds(q * L, L)] for q in range(NQ)]
    lane = jnp.arange(L, dtype=jnp.int32)
    inv = jnp.float32(1.0 / EMB)
    seven = jnp.int32(7)
    one = jnp.int32(1)

    def fire(g, buf):
        rsl = pl.ds(pl.multiple_of(g * L, L), L)
        hslab = lax.shift_left(lax.shift_right_logical(hidx_v[rsl], 3), 3)
        tslab = lax.shift_left(lax.shift_right_logical(tidx_v[rsl], 3), 3)
        aslab = lax.shift_left(lax.shift_right_logical(aidx_v[rsl], 3), 3)
        for i in range(L):
            hs = pl.multiple_of(hslab[i], SL)
            ts = pl.multiple_of(tslab[i], SL)
            as_ = pl.multiple_of(aslab[i], SL)
            pltpu.async_copy(
                ct_hbm.at[pl.ds(hs, SL), :], h_v[buf].at[i], sems[buf])
            pltpu.async_copy(
                ct_hbm.at[pl.ds(ts, SL), :], t_v[buf].at[i], sems[buf])
            pltpu.async_copy(
                at_hbm.at[pl.ds(as_, SL), :], a_v[buf].at[i], sems[buf])

    def drain(buf):
        for i in range(L):
            pltpu.make_async_copy(
                ct_hbm.at[pl.ds(0, SL), :], h_v[buf].at[i], sems[buf]).wait()
            pltpu.make_async_copy(
                ct_hbm.at[pl.ds(0, SL), :], t_v[buf].at[i], sems[buf]).wait()
            pltpu.make_async_copy(
                at_hbm.at[pl.ds(0, SL), :], a_v[buf].at[i], sems[buf]).wait()

    def compute(g, buf):
        row0 = pl.multiple_of(g * L, L)
        rsl = pl.ds(row0, L)
        phv = lax.bitwise_and(hidx_v[rsl], seven)
        ptv = lax.bitwise_and(tidx_v[rsl], seven)
        pav = lax.bitwise_and(aidx_v[rsl], seven)
        for i in range(L):
            ph = phv[i]
            pt = ptv[i]
            pa = pav[i]
            d = None
            for q in range(NQ):
                sl2 = pl.ds(q * L, L)
                hq = h_v[buf][i, ph, sl2]
                tq = t_v[buf][i, pt, sl2]
                aq = a_v[buf][i, pa, sl2]
                dq = jnp.abs(hq + aq - tq + cs[q])
                d = dq if d is None else d + dq
            plsc.store_scatter(
                pbuf_v, [lane, jnp.full((L,), i, jnp.int32)], d)
        acc = pbuf_v[0, :]
        for rr in range(1, L):
            acc = acc + pbuf_v[rr, :]
        out_v[rsl] = acc * inv

    fire(0, 0)

    def body(k, carry):
        g0 = lax.mul(k, jnp.int32(2))
        fire(g0 + 1, 1)
        drain(0)
        compute(g0, 0)
        fire(jnp.minimum(g0 + 2, jnp.int32(G - 1)), 0)
        drain(1)
        compute(g0 + 1, 1)
        return carry

    lax.fori_loop(0, G // 2, body, 0)
    drain(0)

    pltpu.sync_copy(out_v, out_hbm.at[pl.ds(base, BPW)])


def kernel(data, concept_table, act_table, begin, end):
    head = data[:, 0].astype(jnp.int32)
    act = data[:, 1].astype(jnp.int32)
    tail = data[:, 2].astype(jnp.int32)
    cvec = (begin - end).reshape(EMB).astype(jnp.float32)
    return _transe_sc(head, tail, act, concept_table, act_table, cvec)
